# baseline (speedup 1.0000x reference)
import functools

import jax
import jax.numpy as jnp
from jax.experimental import pallas as pl
from jax.experimental.pallas import tpu as pltpu

_EPS = 1e-5
_C = 32


def _banded_weights_all(w1b, w2b, wcb, W, C):
    f32 = jnp.float32

    def compact(wb):
        return wb.reshape(9, W, C, W, C)[:, 0, :, 0, :]

    wk = jnp.stack([compact(w1b), compact(w2b), compact(wcb)])
    wk = wk.reshape(3, 3, 3, C, C).astype(f32)
    shifts = jnp.stack([jnp.eye(W, k=1 - kw, dtype=f32) for kw in range(3)])
    bands = jnp.einsum('kuv,ghkab->ghuavb', shifts, wk)
    bands = bands.reshape(3, 3, W * C, W * C).astype(jnp.bfloat16)
    return bands[0], bands[1], bands[2]


_PB = 16


def _block_kernel(nb, h,
                  x_ref, cx_ref, w1_ref, w2_ref, wc_ref, mavg_ref,
                  g1_ref, b1_ref, g2_ref, b2_ref, bc_ref,
                  res_ref, cpm_ref, pres_ref, pcpm_ref):
    wc = x_ref.shape[-1]
    f32, bf16 = jnp.float32, jnp.bfloat16
    mavg = mavg_ref[...]

    def conv(pad_ref, w_ref):
        acc = jnp.dot(pad_ref[:, _PB - 1:_PB - 1 + h, :].reshape(nb * h, wc),
                      w_ref[0], preferred_element_type=f32)
        for kh in (1, 2):
            acc += jnp.dot(
                pad_ref[:, _PB - 1 + kh:_PB - 1 + kh + h, :].reshape(nb * h, wc),
                w_ref[kh], preferred_element_type=f32)
        return acc

    def inorm(y, g, b):
        y3 = y.reshape(nb, h, wc)
        s1 = jnp.sum(y3, axis=1)
        s2 = jnp.sum(y3 * y3, axis=1)
        st = jnp.dot(jnp.concatenate([s1, s2], axis=0), mavg,
                     preferred_element_type=f32)
        mean, ms = st[:nb], st[nb:]
        scale = g * jax.lax.rsqrt(ms - mean * mean + _EPS)
        shift = b - mean * scale
        return y3 * scale[:, None, :] + shift[:, None, :]

    zrow = jnp.zeros((nb, 1, wc), bf16)
    pres_ref[:, _PB - 1:_PB, :] = zrow
    pres_ref[:, _PB + h:_PB + h + 1, :] = zrow
    pcpm_ref[:, _PB - 1:_PB, :] = zrow
    pcpm_ref[:, _PB + h:_PB + h + 1, :] = zrow

    pcpm_ref[:, _PB:_PB + h, :] = cx_ref[...].astype(bf16)
    yc = conv(pcpm_ref, wc_ref).reshape(nb, h, wc) + bc_ref[...][None, :, :]
    cpm_ref[...] = jnp.maximum(yc, 0.0)

    x = x_ref[...]
    pres_ref[:, _PB:_PB + h, :] = x.astype(bf16)
    y1 = jnp.maximum(inorm(conv(pres_ref, w1_ref), g1_ref[...], b1_ref[...]),
                     0.0)
    pres_ref[:, _PB:_PB + h, :] = y1.astype(bf16)
    y2 = inorm(conv(pres_ref, w2_ref), g2_ref[...], b2_ref[...])
    res_ref[...] = jnp.maximum(x + y2, 0.0)


def kernel(x2d, cx2d, w1b, w2b, wcb, mavg, g1t, b1t, g2t, b2t, bct):
    N, H, WC = x2d.shape
    C = _C
    W = WC // C
    nb = next(b for b in (16, 8, 4, 2, 1) if N % b == 0)
    w1s, w2s, wcs = _banded_weights_all(w1b, w2b, wcb, W, C)
    f32 = jnp.float32

    io_spec = pl.BlockSpec((nb, H, WC), lambda n: (n, 0, 0))

    def const_spec(a):
        nd = a.ndim
        idx = lambda n, _nd=nd: (0,) * _nd
        try:
            return pl.BlockSpec(a.shape, idx, pipeline_mode=pl.Buffered(1))
        except Exception:
            return pl.BlockSpec(a.shape, idx)

    res, cpm = pl.pallas_call(
        functools.partial(_block_kernel, nb, H),
        out_shape=(jax.ShapeDtypeStruct((N, H, WC), f32),
                   jax.ShapeDtypeStruct((N, H, WC), f32)),
        grid=(N // nb,),
        in_specs=[io_spec, io_spec,
                  const_spec(w1s), const_spec(w2s), const_spec(wcs),
                  const_spec(mavg), const_spec(g1t), const_spec(b1t),
                  const_spec(g2t), const_spec(b2t), const_spec(bct)],
        out_specs=(io_spec, io_spec),
        scratch_shapes=[pltpu.VMEM((nb, H + 2 * _PB, WC), jnp.bfloat16),
                        pltpu.VMEM((nb, H + 2 * _PB, WC), jnp.bfloat16)],
        compiler_params=pltpu.CompilerParams(
            dimension_semantics=("parallel",),
            vmem_limit_bytes=64 * 1024 * 1024),
    )(x2d, cx2d, w1s, w2s, wcs, mavg, g1t, b1t, g2t, b2t, bct)
    return res, cpm


# speedup vs baseline: 1.1983x; 1.1983x over previous
import functools

import jax
import jax.numpy as jnp
from jax.experimental import pallas as pl
from jax.experimental.pallas import tpu as pltpu

_EPS = 1e-5
_C = 32


def _banded_weights_all(w1b, w2b, wcb, W, C):
    f32 = jnp.float32

    bf16 = jnp.bfloat16
    shifts = [jnp.eye(W, k=1 - kw, dtype=f32).astype(bf16)[:, None, :, None]
              for kw in range(3)]

    def bands(wb):
        wk = wb.reshape(9, W, C, W, C)[:, 0, :, 0, :]
        out = []
        for kh in range(3):
            acc = None
            for kw in range(3):
                t = shifts[kw] * wk[kh * 3 + kw][None, :, None, :]
                acc = t if acc is None else acc + t
            out.append(acc.reshape(W * C, W * C))
        return out

    return bands(w1b), bands(w2b), bands(wcb)


_PB = 16


def _block_kernel(nb, h,
                  x_ref, cx_ref, w10_ref, w11_ref, w12_ref,
                  w20_ref, w21_ref, w22_ref, wc0_ref, wc1_ref, wc2_ref,
                  mavg_ref, g1_ref, b1_ref, g2_ref, b2_ref, bc_ref,
                  res_ref, cpm_ref, pres_ref, pcpm_ref):
    wc = x_ref.shape[-1]
    f32, bf16 = jnp.float32, jnp.bfloat16
    mavg = mavg_ref[...]

    def conv(pad_ref, w_refs):
        acc = jnp.dot(pad_ref[:, _PB - 1:_PB - 1 + h, :].reshape(nb * h, wc),
                      w_refs[0][...], preferred_element_type=f32)
        for kh in (1, 2):
            acc += jnp.dot(
                pad_ref[:, _PB - 1 + kh:_PB - 1 + kh + h, :].reshape(nb * h, wc),
                w_refs[kh][...], preferred_element_type=f32)
        return acc

    def inorm(y, g, b):
        y3 = y.reshape(nb, h, wc)
        s1 = jnp.sum(y3, axis=1)
        s2 = jnp.sum(y3 * y3, axis=1)
        st = jnp.dot(jnp.concatenate([s1, s2], axis=0), mavg,
                     preferred_element_type=f32)
        mean, ms = st[:nb], st[nb:]
        scale = g * jax.lax.rsqrt(ms - mean * mean + _EPS)
        shift = b - mean * scale
        return y3 * scale[:, None, :] + shift[:, None, :]

    zrow = jnp.zeros((nb, 1, wc), bf16)
    pres_ref[:, _PB - 1:_PB, :] = zrow
    pres_ref[:, _PB + h:_PB + h + 1, :] = zrow
    pcpm_ref[:, _PB - 1:_PB, :] = zrow
    pcpm_ref[:, _PB + h:_PB + h + 1, :] = zrow

    pcpm_ref[:, _PB:_PB + h, :] = cx_ref[...].astype(bf16)
    yc = conv(pcpm_ref, (wc0_ref, wc1_ref, wc2_ref)).reshape(nb, h, wc)
    cpm_ref[...] = jnp.maximum(yc + bc_ref[...][None, :, :], 0.0)

    x = x_ref[...]
    pres_ref[:, _PB:_PB + h, :] = x.astype(bf16)
    y1 = jnp.maximum(inorm(conv(pres_ref, (w10_ref, w11_ref, w12_ref)),
                           g1_ref[...], b1_ref[...]), 0.0)
    pres_ref[:, _PB:_PB + h, :] = y1.astype(bf16)
    y2 = inorm(conv(pres_ref, (w20_ref, w21_ref, w22_ref)),
               g2_ref[...], b2_ref[...])
    res_ref[...] = jnp.maximum(x + y2, 0.0)


def kernel(x2d, cx2d, w1b, w2b, wcb, mavg, g1t, b1t, g2t, b2t, bct):
    N, H, WC = x2d.shape
    C = _C
    W = WC // C
    nb = next(b for b in (16, 8, 4, 2, 1) if N % b == 0)
    w1s, w2s, wcs = _banded_weights_all(w1b, w2b, wcb, W, C)
    f32 = jnp.float32

    io_spec = pl.BlockSpec((nb, H, WC), lambda n: (n, 0, 0))

    def const_spec(a):
        nd = a.ndim
        idx = lambda n, _nd=nd: (0,) * _nd
        try:
            return pl.BlockSpec(a.shape, idx, pipeline_mode=pl.Buffered(1))
        except Exception:
            return pl.BlockSpec(a.shape, idx)

    res, cpm = pl.pallas_call(
        functools.partial(_block_kernel, nb, H),
        out_shape=(jax.ShapeDtypeStruct((N, H, WC), f32),
                   jax.ShapeDtypeStruct((N, H, WC), f32)),
        grid=(N // nb,),
        in_specs=[io_spec, io_spec]
                 + [const_spec(w) for w in (*w1s, *w2s, *wcs)]
                 + [const_spec(a) for a in (mavg, g1t, b1t, g2t, b2t, bct)],
        out_specs=(io_spec, io_spec),
        scratch_shapes=[pltpu.VMEM((nb, H + 2 * _PB, WC), jnp.bfloat16),
                        pltpu.VMEM((nb, H + 2 * _PB, WC), jnp.bfloat16)],
        compiler_params=pltpu.CompilerParams(
            dimension_semantics=("parallel",),
            vmem_limit_bytes=64 * 1024 * 1024),
    )(x2d, cx2d, *w1s, *w2s, *wcs, mavg, g1t, b1t, g2t, b2t, bct)
    return res, cpm


# speedup vs baseline: 3.6366x; 3.0348x over previous
import functools

import jax
import jax.numpy as jnp
from jax.experimental import pallas as pl
from jax.experimental.pallas import tpu as pltpu

_EPS = 1e-5
_C = 32


def _banded_weights_all(w1b, w2b, wcb, W, C):
    WC = W * C

    def bands(wb):
        out = []
        for kh in range(3):
            acc = None
            for kw in range(3):
                k = kh * 3 + kw
                T = jax.lax.slice_in_dim(wb, k * WC, (k + 1) * WC, axis=0)
                s = (kw - 1) * C
                if s < 0:
                    Tb = jnp.pad(T[:, :WC + s], ((0, 0), (-s, 0)))
                elif s > 0:
                    Tb = jnp.pad(T[:, s:], ((0, 0), (0, s)))
                else:
                    Tb = T
                acc = Tb if acc is None else acc + Tb
            out.append(acc)
        return out

    return bands(w1b), bands(w2b), bands(wcb)


_PB = 16


def _block_kernel(nb, h,
                  x_ref, cx_ref, w10_ref, w11_ref, w12_ref,
                  w20_ref, w21_ref, w22_ref, wc0_ref, wc1_ref, wc2_ref,
                  mavg_ref, g1_ref, b1_ref, g2_ref, b2_ref, bc_ref,
                  res_ref, cpm_ref, pres_ref, pcpm_ref):
    wc = x_ref.shape[-1]
    f32, bf16 = jnp.float32, jnp.bfloat16
    mavg = mavg_ref[...]

    def conv(pad_ref, w_refs):
        acc = jnp.dot(pad_ref[:, _PB - 1:_PB - 1 + h, :].reshape(nb * h, wc),
                      w_refs[0][...], preferred_element_type=f32)
        for kh in (1, 2):
            acc += jnp.dot(
                pad_ref[:, _PB - 1 + kh:_PB - 1 + kh + h, :].reshape(nb * h, wc),
                w_refs[kh][...], preferred_element_type=f32)
        return acc

    def inorm(y, g, b):
        y3 = y.reshape(nb, h, wc)
        s1 = jnp.sum(y3, axis=1)
        s2 = jnp.sum(y3 * y3, axis=1)
        st = jnp.dot(jnp.concatenate([s1, s2], axis=0), mavg,
                     preferred_element_type=f32)
        mean, ms = st[:nb], st[nb:]
        scale = g * jax.lax.rsqrt(ms - mean * mean + _EPS)
        shift = b - mean * scale
        return y3 * scale[:, None, :] + shift[:, None, :]

    zrow = jnp.zeros((nb, 1, wc), bf16)
    pres_ref[:, _PB - 1:_PB, :] = zrow
    pres_ref[:, _PB + h:_PB + h + 1, :] = zrow
    pcpm_ref[:, _PB - 1:_PB, :] = zrow
    pcpm_ref[:, _PB + h:_PB + h + 1, :] = zrow

    pcpm_ref[:, _PB:_PB + h, :] = cx_ref[...].astype(bf16)
    yc = conv(pcpm_ref, (wc0_ref, wc1_ref, wc2_ref)).reshape(nb, h, wc)
    cpm_ref[...] = jnp.maximum(yc + bc_ref[...][None, :, :], 0.0)

    x = x_ref[...]
    pres_ref[:, _PB:_PB + h, :] = x.astype(bf16)
    y1 = jnp.maximum(inorm(conv(pres_ref, (w10_ref, w11_ref, w12_ref)),
                           g1_ref[...], b1_ref[...]), 0.0)
    pres_ref[:, _PB:_PB + h, :] = y1.astype(bf16)
    y2 = inorm(conv(pres_ref, (w20_ref, w21_ref, w22_ref)),
               g2_ref[...], b2_ref[...])
    res_ref[...] = jnp.maximum(x + y2, 0.0)


def kernel(x2d, cx2d, w1b, w2b, wcb, mavg, g1t, b1t, g2t, b2t, bct):
    N, H, WC = x2d.shape
    C = _C
    W = WC // C
    nb = next(b for b in (16, 8, 4, 2, 1) if N % b == 0)
    w1s, w2s, wcs = _banded_weights_all(w1b, w2b, wcb, W, C)
    f32 = jnp.float32

    io_spec = pl.BlockSpec((nb, H, WC), lambda n: (n, 0, 0))

    def const_spec(a):
        nd = a.ndim
        idx = lambda n, _nd=nd: (0,) * _nd
        try:
            return pl.BlockSpec(a.shape, idx, pipeline_mode=pl.Buffered(1))
        except Exception:
            return pl.BlockSpec(a.shape, idx)

    res, cpm = pl.pallas_call(
        functools.partial(_block_kernel, nb, H),
        out_shape=(jax.ShapeDtypeStruct((N, H, WC), f32),
                   jax.ShapeDtypeStruct((N, H, WC), f32)),
        grid=(N // nb,),
        in_specs=[io_spec, io_spec]
                 + [const_spec(w) for w in (*w1s, *w2s, *wcs)]
                 + [const_spec(a) for a in (mavg, g1t, b1t, g2t, b2t, bct)],
        out_specs=(io_spec, io_spec),
        scratch_shapes=[pltpu.VMEM((nb, H + 2 * _PB, WC), jnp.bfloat16),
                        pltpu.VMEM((nb, H + 2 * _PB, WC), jnp.bfloat16)],
        compiler_params=pltpu.CompilerParams(
            dimension_semantics=("parallel",),
            vmem_limit_bytes=64 * 1024 * 1024),
    )(x2d, cx2d, *w1s, *w2s, *wcs, mavg, g1t, b1t, g2t, b2t, bct)
    return res, cpm


# speedup vs baseline: 3.7133x; 1.0211x over previous
import functools

import jax
import jax.numpy as jnp
from jax.experimental import pallas as pl
from jax.experimental.pallas import tpu as pltpu

_EPS = 1e-5
_C = 32


def _banded_weights_all(w1b, w2b, wcb, W, C):
    WC = W * C

    def bands(wb):
        out = []
        for kh in range(3):
            acc = None
            for kw in range(3):
                k = kh * 3 + kw
                T = jax.lax.slice_in_dim(wb, k * WC, (k + 1) * WC, axis=0)
                s = (kw - 1) * C
                if s < 0:
                    Tb = jnp.pad(T[:, :WC + s], ((0, 0), (-s, 0)))
                elif s > 0:
                    Tb = jnp.pad(T[:, s:], ((0, 0), (0, s)))
                else:
                    Tb = T
                acc = Tb if acc is None else acc + Tb
            out.append(acc)
        return out

    return bands(w1b), bands(w2b), bands(wcb)


_PB = 16


def _block_kernel(nb, h,
                  x_ref, cx_ref, w10_ref, w11_ref, w12_ref,
                  w20_ref, w21_ref, w22_ref, wc0_ref, wc1_ref, wc2_ref,
                  mavg_ref, g1_ref, b1_ref, g2_ref, b2_ref, bc_ref,
                  res_ref, cpm_ref, pres_ref, pcpm_ref):
    wc = x_ref.shape[-1]
    f32, bf16 = jnp.float32, jnp.bfloat16
    mavg = mavg_ref[...]

    def conv(pad_ref, w_refs):
        acc = jnp.dot(pad_ref[:, _PB - 1:_PB - 1 + h, :].reshape(nb * h, wc),
                      w_refs[0][...], preferred_element_type=f32)
        for kh in (1, 2):
            acc += jnp.dot(
                pad_ref[:, _PB - 1 + kh:_PB - 1 + kh + h, :].reshape(nb * h, wc),
                w_refs[kh][...], preferred_element_type=f32)
        return acc

    def inorm(y, g, b):
        y3 = y.reshape(nb, h, wc)
        s1 = jnp.sum(y3, axis=1)
        s2 = jnp.sum(y3 * y3, axis=1)
        st = jnp.dot(jnp.concatenate([s1, s2], axis=0), mavg,
                     preferred_element_type=f32)
        mean, ms = st[:nb], st[nb:]
        scale = g * jax.lax.rsqrt(ms - mean * mean + _EPS)
        shift = b - mean * scale
        return y3 * scale[:, None, :] + shift[:, None, :]

    zrow = jnp.zeros((nb, 1, wc), bf16)
    pres_ref[:, _PB - 1:_PB, :] = zrow
    pres_ref[:, _PB + h:_PB + h + 1, :] = zrow
    pcpm_ref[:, _PB - 1:_PB, :] = zrow
    pcpm_ref[:, _PB + h:_PB + h + 1, :] = zrow

    x = x_ref[...]
    pres_ref[:, _PB:_PB + h, :] = x.astype(bf16)
    pcpm_ref[:, _PB:_PB + h, :] = cx_ref[...].astype(bf16)
    y1 = jnp.maximum(inorm(conv(pres_ref, (w10_ref, w11_ref, w12_ref)),
                           g1_ref[...], b1_ref[...]), 0.0)
    pres_ref[:, _PB:_PB + h, :] = y1.astype(bf16)
    c2 = conv(pres_ref, (w20_ref, w21_ref, w22_ref))
    yc = conv(pcpm_ref, (wc0_ref, wc1_ref, wc2_ref)).reshape(nb, h, wc)
    cpm_ref[...] = jnp.maximum(yc + bc_ref[...][None, :, :], 0.0)
    y2 = inorm(c2, g2_ref[...], b2_ref[...])
    res_ref[...] = jnp.maximum(x + y2, 0.0)


def kernel(x2d, cx2d, w1b, w2b, wcb, mavg, g1t, b1t, g2t, b2t, bct):
    N, H, WC = x2d.shape
    C = _C
    W = WC // C
    nb = next(b for b in (16, 8, 4, 2, 1) if N % b == 0)
    w1s, w2s, wcs = _banded_weights_all(w1b, w2b, wcb, W, C)
    f32 = jnp.float32

    io_spec = pl.BlockSpec((nb, H, WC), lambda n: (n, 0, 0))

    def const_spec(a):
        nd = a.ndim
        idx = lambda n, _nd=nd: (0,) * _nd
        try:
            return pl.BlockSpec(a.shape, idx, pipeline_mode=pl.Buffered(1))
        except Exception:
            return pl.BlockSpec(a.shape, idx)

    res, cpm = pl.pallas_call(
        functools.partial(_block_kernel, nb, H),
        out_shape=(jax.ShapeDtypeStruct((N, H, WC), f32),
                   jax.ShapeDtypeStruct((N, H, WC), f32)),
        grid=(N // nb,),
        in_specs=[io_spec, io_spec]
                 + [const_spec(w) for w in (*w1s, *w2s, *wcs)]
                 + [const_spec(a) for a in (mavg, g1t, b1t, g2t, b2t, bct)],
        out_specs=(io_spec, io_spec),
        scratch_shapes=[pltpu.VMEM((nb, H + 2 * _PB, WC), jnp.bfloat16),
                        pltpu.VMEM((nb, H + 2 * _PB, WC), jnp.bfloat16)],
        compiler_params=pltpu.CompilerParams(
            dimension_semantics=("parallel",),
            vmem_limit_bytes=64 * 1024 * 1024),
    )(x2d, cx2d, *w1s, *w2s, *wcs, mavg, g1t, b1t, g2t, b2t, bct)
    return res, cpm


# speedup vs baseline: 4.1157x; 1.1084x over previous
import functools

import jax
import jax.numpy as jnp
from jax.experimental import pallas as pl
from jax.experimental.pallas import tpu as pltpu

_EPS = 1e-5
_C = 32
_PB = 16


def _block_kernel(nb, h, W, C,
                  x_ref, cx_ref, w1b_ref, w2b_ref, wcb_ref,
                  mavg_ref, g1_ref, b1_ref, g2_ref, b2_ref, bc_ref,
                  res_ref, cpm_ref, pres_ref, pcpm_ref, wband_ref):
    wc = x_ref.shape[-1]
    f32, bf16 = jnp.float32, jnp.bfloat16
    mavg = mavg_ref[...]

    @pl.when(pl.program_id(1) == 0)
    def _fold_weights():
        zc = jnp.zeros((wc, C), bf16)
        for c, wb_ref in enumerate((w1b_ref, w2b_ref, wcb_ref)):
            for kh in range(3):
                t0 = wb_ref[pl.ds((kh * 3 + 0) * wc, wc), :]
                t1 = wb_ref[pl.ds((kh * 3 + 1) * wc, wc), :]
                t2 = wb_ref[pl.ds((kh * 3 + 2) * wc, wc), :]
                band = (t1
                        + jnp.concatenate([zc, t0[:, :wc - C]], axis=1)
                        + jnp.concatenate([t2[:, C:], zc], axis=1))
                wband_ref[c * 3 + kh] = band

    def conv(pad_ref, base):
        acc = jnp.dot(pad_ref[:, _PB - 1:_PB - 1 + h, :].reshape(nb * h, wc),
                      wband_ref[base], preferred_element_type=f32)
        for kh in (1, 2):
            acc += jnp.dot(
                pad_ref[:, _PB - 1 + kh:_PB - 1 + kh + h, :].reshape(nb * h, wc),
                wband_ref[base + kh], preferred_element_type=f32)
        return acc

    def inorm(y, g, b):
        y3 = y.reshape(nb, h, wc)
        s1 = jnp.sum(y3, axis=1)
        s2 = jnp.sum(y3 * y3, axis=1)
        st = jnp.dot(jnp.concatenate([s1, s2], axis=0), mavg,
                     preferred_element_type=f32)
        mean, ms = st[:nb], st[nb:]
        scale = g * jax.lax.rsqrt(ms - mean * mean + _EPS)
        shift = b - mean * scale
        return y3 * scale[:, None, :] + shift[:, None, :]

    zrow = jnp.zeros((nb, 1, wc), bf16)
    pres_ref[:, _PB - 1:_PB, :] = zrow
    pres_ref[:, _PB + h:_PB + h + 1, :] = zrow
    pcpm_ref[:, _PB - 1:_PB, :] = zrow
    pcpm_ref[:, _PB + h:_PB + h + 1, :] = zrow

    x = x_ref[...]
    pres_ref[:, _PB:_PB + h, :] = x.astype(bf16)
    pcpm_ref[:, _PB:_PB + h, :] = cx_ref[...].astype(bf16)
    y1 = jnp.maximum(inorm(conv(pres_ref, 0), g1_ref[...], b1_ref[...]), 0.0)
    pres_ref[:, _PB:_PB + h, :] = y1.astype(bf16)
    c2 = conv(pres_ref, 3)
    yc = conv(pcpm_ref, 6).reshape(nb, h, wc)
    cpm_ref[...] = jnp.maximum(yc + bc_ref[...][None, :, :], 0.0)
    y2 = inorm(c2, g2_ref[...], b2_ref[...])
    res_ref[...] = jnp.maximum(x + y2, 0.0)


def kernel(x2d, cx2d, w1b, w2b, wcb, mavg, g1t, b1t, g2t, b2t, bct):
    N, H, WC = x2d.shape
    C = _C
    W = WC // C
    f32 = jnp.float32
    nb = next(b for b in (16, 8, 4, 2, 1) if N % b == 0)
    steps = N // nb
    ncore = 2 if steps % 2 == 0 else 1
    inner = steps // ncore

    io_spec = pl.BlockSpec((nb, H, WC), lambda o, i, _g=inner: (o * _g + i, 0, 0))

    def const_spec(a):
        nd = a.ndim
        idx = lambda o, i, _nd=nd: (0,) * _nd
        try:
            return pl.BlockSpec(a.shape, idx, pipeline_mode=pl.Buffered(1))
        except Exception:
            return pl.BlockSpec(a.shape, idx)

    res, cpm = pl.pallas_call(
        functools.partial(_block_kernel, nb, H, W, C),
        out_shape=(jax.ShapeDtypeStruct((N, H, WC), f32),
                   jax.ShapeDtypeStruct((N, H, WC), f32)),
        grid=(ncore, inner),
        in_specs=[io_spec, io_spec]
                 + [const_spec(a) for a in (w1b, w2b, wcb, mavg,
                                            g1t, b1t, g2t, b2t, bct)],
        out_specs=(io_spec, io_spec),
        scratch_shapes=[pltpu.VMEM((nb, H + 2 * _PB, WC), jnp.bfloat16),
                        pltpu.VMEM((nb, H + 2 * _PB, WC), jnp.bfloat16),
                        pltpu.VMEM((9, WC, WC), jnp.bfloat16)],
        compiler_params=pltpu.CompilerParams(
            dimension_semantics=("parallel", "arbitrary"),
            vmem_limit_bytes=64 * 1024 * 1024),
    )(x2d, cx2d, w1b, w2b, wcb, mavg, g1t, b1t, g2t, b2t, bct)
    return res, cpm
